# baseline (speedup 1.0000x reference)
import jax
import jax.numpy as jnp
from jax import lax
from jax.experimental import pallas as pl
from jax.experimental.pallas import tpu as pltpu
from jax.experimental.pallas import tpu_sc as plsc

NUM_OBJECTS = 1000000
EMBED_DIM = 64
BATCH = 16384

_info = plsc.get_sparse_core_info()
_NC, _NS, _L = _info.num_cores, _info.num_subcores, _info.num_lanes
_NW = _NC * _NS
_B_PER_W = BATCH // _NW
_CHUNK = 128
_NCHUNK = _B_PER_W // _CHUNK


def _rsqrt16(s):
    bits = lax.bitcast_convert_type(s, jnp.int32)
    y = lax.bitcast_convert_type(jnp.int32(0x5F3759DF) - (bits >> 1),
                                 jnp.float32)
    half = s * 0.5
    for _ in range(3):
        y = y * (1.5 - half * y * y)
    return y


def _sc_body(tab_hbm, idx_hbm, out_hbm, idx_v, rows_v, sem):
    wid = lax.axis_index("s") * _NC + lax.axis_index("c")
    base = wid * _B_PER_W

    for g in range(_NCHUNK):
        pltpu.sync_copy(idx_hbm.at[pl.ds(base + g * _CHUNK, _CHUNK)],
                        idx_v.at[g])

    for g in range(_NCHUNK):
        pltpu.async_copy(tab_hbm.at[idx_v.at[g]],
                         rows_v.at[pl.ds(g * _CHUNK, _CHUNK)], sem)
    pltpu.make_async_copy(tab_hbm.at[pl.ds(0, _B_PER_W)], rows_v, sem).wait()

    lanes = lax.iota(jnp.int32, _L)

    def norm_body(g, _):
        row_idx = g * _L + lanes
        ss = jnp.zeros((_L,), jnp.float32)
        for c in range(EMBED_DIM):
            col = jnp.full((_L,), c, jnp.int32)
            v = plsc.load_gather(rows_v, [row_idx, col])
            ss = ss + v * v
        inv = _rsqrt16(jnp.maximum(ss, 1e-12))
        for c in range(EMBED_DIM):
            col = jnp.full((_L,), c, jnp.int32)
            v = plsc.load_gather(rows_v, [row_idx, col])
            plsc.store_scatter(rows_v, [row_idx, col], v * inv)
        return 0

    lax.fori_loop(0, _B_PER_W // _L, norm_body, 0)
    pltpu.sync_copy(rows_v, out_hbm.at[pl.ds(base, _B_PER_W)])


@jax.jit
def _encode(ids, table):
    mesh = plsc.VectorSubcoreMesh(core_axis_name="c", subcore_axis_name="s")
    call = pl.kernel(
        _sc_body,
        mesh=mesh,
        out_type=jax.ShapeDtypeStruct((BATCH, EMBED_DIM), jnp.float32),
        scratch_types=[
            pltpu.VMEM((_NCHUNK, _CHUNK), jnp.int32),
            pltpu.VMEM((_B_PER_W, EMBED_DIM), jnp.float32),
            pltpu.SemaphoreType.DMA,
        ],
        compiler_params=pltpu.CompilerParams(needs_layout_passes=False,
                                             use_tc_tiling_on_sc=False),
    )
    return call(table, ids.astype(jnp.int32))


def kernel(ids, table):
    return _encode(ids, table)


# speedup vs baseline: 1.1139x; 1.1139x over previous
import jax
import jax.numpy as jnp
from jax import lax
from jax.experimental import pallas as pl
from jax.experimental.pallas import tpu as pltpu
from jax.experimental.pallas import tpu_sc as plsc

NUM_OBJECTS = 1000000
EMBED_DIM = 64
BATCH = 16384

_info = plsc.get_sparse_core_info()
_NC, _NS, _L = _info.num_cores, _info.num_subcores, _info.num_lanes
_NW = _NC * _NS
_B_PER_W = BATCH // _NW
_CHUNK = 128
_NCHUNK = _B_PER_W // _CHUNK


def _rsqrt16(s):
    bits = lax.bitcast_convert_type(s, jnp.int32)
    y = lax.bitcast_convert_type(jnp.int32(0x5F3759DF) - (bits >> 1),
                                 jnp.float32)
    half = s * 0.5
    for _ in range(3):
        y = y * (1.5 - half * y * y)
    return y


_PADDED = 2 * EMBED_DIM


def _sc_body(tab_hbm, idx_hbm, out_hbm, idx_v, rows_v, sem):
    wid = lax.axis_index("s") * _NC + lax.axis_index("c")
    base = wid * _B_PER_W

    for g in range(_NCHUNK):
        pltpu.sync_copy(idx_hbm.at[pl.ds(base + g * _CHUNK, _CHUNK)],
                        idx_v.at[g])

    for g in range(_NCHUNK):
        pltpu.async_copy(tab_hbm.at[idx_v.at[g]],
                         rows_v.at[pl.ds(g * _CHUNK, _CHUNK)], sem)
    pltpu.make_async_copy(tab_hbm.at[pl.ds(0, _B_PER_W)], rows_v, sem).wait()

    lanes = lax.iota(jnp.int32, _L)

    def norm_body(g, _):
        row_idx = g * _L + lanes
        ss = jnp.zeros((_L,), jnp.float32)
        for c in range(EMBED_DIM):
            col = jnp.full((_L,), c, jnp.int32)
            v = plsc.load_gather(rows_v, [row_idx, col])
            ss = ss + v * v
        inv = _rsqrt16(jnp.maximum(ss, 1e-12))
        for c in range(EMBED_DIM):
            col = jnp.full((_L,), c, jnp.int32)
            v = plsc.load_gather(rows_v, [row_idx, col])
            plsc.store_scatter(rows_v, [row_idx, col], v * inv)
        return 0

    lax.fori_loop(0, _B_PER_W // _L, norm_body, 0)
    pltpu.sync_copy(rows_v, out_hbm.at[pl.ds(base, _B_PER_W)])


@jax.jit
def _encode(ids, table):
    mesh = plsc.VectorSubcoreMesh(core_axis_name="c", subcore_axis_name="s")
    call = pl.kernel(
        _sc_body,
        mesh=mesh,
        out_type=jax.ShapeDtypeStruct((BATCH, _PADDED), jnp.float32),
        scratch_types=[
            pltpu.VMEM((_NCHUNK, _CHUNK), jnp.int32),
            pltpu.VMEM((_B_PER_W, _PADDED), jnp.float32),
            pltpu.SemaphoreType.DMA,
        ],
        compiler_params=pltpu.CompilerParams(needs_layout_passes=False),
    )
    tab128 = jnp.concatenate(
        [table, jnp.zeros((NUM_OBJECTS, EMBED_DIM), jnp.float32)], axis=1)
    return call(tab128, ids.astype(jnp.int32))[:, :EMBED_DIM]


def kernel(ids, table):
    return _encode(ids, table)


# speedup vs baseline: 1.6016x; 1.4378x over previous
import jax
import jax.numpy as jnp
from jax import lax
from jax.experimental import pallas as pl
from jax.experimental.pallas import tpu as pltpu
from jax.experimental.pallas import tpu_sc as plsc

NUM_OBJECTS = 1000000
EMBED_DIM = 64
BATCH = 16384

_info = plsc.get_sparse_core_info()
_NC, _NS, _L = _info.num_cores, _info.num_subcores, _info.num_lanes
_NW = _NC * _NS
_B_PER_W = BATCH // _NW
_CHUNK = 64
_NCHUNK = _B_PER_W // _CHUNK


def _rsqrt16(s):
    bits = lax.bitcast_convert_type(s, jnp.int32)
    y = lax.bitcast_convert_type(jnp.int32(0x5F3759DF) - (bits >> 1),
                                 jnp.float32)
    half = s * 0.5
    for _ in range(3):
        y = y * (1.5 - half * y * y)
    return y


def _sc_body(tab_hbm, idx_hbm, out_hbm, idxf_v, blk_v, stage_v, sem):
    wid = lax.axis_index("s") * _NC + lax.axis_index("c")
    base = wid * _B_PER_W
    pltpu.sync_copy(idx_hbm.at[pl.ds(base, _B_PER_W)], idxf_v)
    lanes = lax.iota(jnp.int32, _L)

    for ch in range(_NCHUNK):
        def fire_body(g, _):
            rvec = idxf_v[pl.ds(ch * _CHUNK + g * _L, _L)]
            for j in range(_L):
                r = rvec[j]
                b8 = pl.multiple_of((r >> 3) * 8, 8)
                pltpu.async_copy(tab_hbm.at[pl.ds(b8, 8), :],
                                 blk_v.at[pl.ds((g * _L + j) * 8, 8)], sem)
            return 0

        lax.fori_loop(0, _CHUNK // _L, fire_body, 0)
        pltpu.make_async_copy(tab_hbm.at[pl.ds(0, 8 * _CHUNK)], blk_v,
                              sem).wait()

        def norm_body(g, _):
            rvec = idxf_v[pl.ds(ch * _CHUNK + g * _L, _L)]
            row_idx = (g * _L + lanes) * 8 + (rvec & 7)
            ss = jnp.zeros((_L,), jnp.float32)
            for c in range(EMBED_DIM):
                col = jnp.full((_L,), c, jnp.int32)
                v = plsc.load_gather(blk_v, [row_idx, col])
                ss = ss + v * v
            inv = _rsqrt16(jnp.maximum(ss, 1e-12))
            dst = (ch % 2) * _CHUNK + g * _L + lanes
            for c in range(EMBED_DIM):
                col = jnp.full((_L,), c, jnp.int32)
                v = plsc.load_gather(blk_v, [row_idx, col])
                plsc.store_scatter(stage_v, [col, dst], v * inv)
            return 0

        lax.fori_loop(0, _CHUNK // _L, norm_body, 0)
        if ch % 2 == 1:
            pltpu.sync_copy(
                stage_v,
                out_hbm.at[:, pl.ds(base + (ch // 2) * 2 * _CHUNK,
                                    2 * _CHUNK)])


@jax.jit
def _encode(ids, table):
    mesh = plsc.VectorSubcoreMesh(core_axis_name="c", subcore_axis_name="s")
    call = pl.kernel(
        _sc_body,
        mesh=mesh,
        out_type=jax.ShapeDtypeStruct((EMBED_DIM, BATCH), jnp.float32),
        scratch_types=[
            pltpu.VMEM((_B_PER_W,), jnp.int32),
            pltpu.VMEM((8 * _CHUNK, EMBED_DIM), jnp.float32),
            pltpu.VMEM((EMBED_DIM, 2 * _CHUNK), jnp.float32),
            pltpu.SemaphoreType.DMA,
        ],
        compiler_params=pltpu.CompilerParams(needs_layout_passes=False),
    )
    return call(table, ids.astype(jnp.int32)).T


def kernel(ids, table):
    return _encode(ids, table)
